# baseline (speedup 1.0000x reference)
import jax
import jax.numpy as jnp
from jax.experimental import pallas as pl
from jax.experimental.pallas import tpu as pltpu

_BM = 40
_NSTREAMS = 5


def _gcn_kernel(*refs):
    adj_refs = refs[:_NSTREAMS]
    x_ref, w_ref, b_ref, out_ref, s_ref = refs[_NSTREAMS:]

    @pl.when(pl.program_id(0) == 0)
    def _():
        s_ref[...] = jnp.dot(
            x_ref[...], w_ref[...], preferred_element_type=jnp.float32
        )

    bm = adj_refs[0].shape[0]
    for j in range(_NSTREAMS):
        acc = jnp.dot(
            adj_refs[j][...], s_ref[...], preferred_element_type=jnp.float32
        )
        out_ref[j * bm : (j + 1) * bm, :] = jnp.maximum(acc + b_ref[...], 0.0)


def kernel(x, adj, W, b):
    n, nfeat = x.shape
    nhid = W.shape[1]

    bm = _BM
    k = _NSTREAMS
    adj_specs = [
        pl.BlockSpec((bm, n), lambda i, j=j: (k * i + j, 0)) for j in range(k)
    ]
    out = pl.pallas_call(
        _gcn_kernel,
        grid=(n // (k * bm),),
        in_specs=adj_specs
        + [
            pl.BlockSpec((n, nfeat), lambda i: (0, 0)),
            pl.BlockSpec((nfeat, nhid), lambda i: (0, 0)),
            pl.BlockSpec((1, nhid), lambda i: (0, 0)),
        ],
        out_specs=pl.BlockSpec((k * bm, nhid), lambda i: (i, 0)),
        out_shape=jax.ShapeDtypeStruct((n, nhid), jnp.float32),
        scratch_shapes=[pltpu.VMEM((n, nhid), jnp.float32)],
    )(*([adj] * k), x, W, b.reshape(1, nhid))
    return out


# speedup vs baseline: 1.6052x; 1.6052x over previous
import jax
import jax.numpy as jnp
from jax.experimental import pallas as pl
from jax.experimental.pallas import tpu as pltpu

_BM = 80
_NSTREAMS = 5


def _gcn_kernel(*refs):
    adj_refs = refs[:_NSTREAMS]
    x_ref, w_ref, b_ref, out_ref, s_ref = refs[_NSTREAMS:]

    @pl.when(pl.program_id(0) == 0)
    def _():
        s_ref[...] = jnp.dot(
            x_ref[...], w_ref[...], preferred_element_type=jnp.float32
        )

    bm = adj_refs[0].shape[0]
    for j in range(_NSTREAMS):
        acc = jnp.dot(
            adj_refs[j][...], s_ref[...], preferred_element_type=jnp.float32
        )
        out_ref[j * bm : (j + 1) * bm, :] = jnp.maximum(acc + b_ref[...], 0.0)


def kernel(x, adj, W, b):
    n, nfeat = x.shape
    nhid = W.shape[1]

    bm = _BM
    k = _NSTREAMS
    adj_specs = [
        pl.BlockSpec((bm, n), lambda i, j=j: (k * i + j, 0)) for j in range(k)
    ]
    out = pl.pallas_call(
        _gcn_kernel,
        grid=(n // (k * bm),),
        in_specs=adj_specs
        + [
            pl.BlockSpec((n, nfeat), lambda i: (0, 0)),
            pl.BlockSpec((nfeat, nhid), lambda i: (0, 0)),
            pl.BlockSpec((1, nhid), lambda i: (0, 0)),
        ],
        out_specs=pl.BlockSpec((k * bm, nhid), lambda i: (i, 0)),
        out_shape=jax.ShapeDtypeStruct((n, nhid), jnp.float32),
        scratch_shapes=[pltpu.VMEM((n, nhid), jnp.float32)],
    )(*([adj] * k), x, W, b.reshape(1, nhid))
    return out


# speedup vs baseline: 1.6519x; 1.0291x over previous
import jax
import jax.numpy as jnp
from jax.experimental import pallas as pl
from jax.experimental.pallas import tpu as pltpu

_BM = 200
_NSTREAMS = 2


def _gcn_kernel(*refs):
    adj_refs = refs[:_NSTREAMS]
    x_ref, w_ref, b_ref, out_ref, s_ref = refs[_NSTREAMS:]

    @pl.when(pl.program_id(0) == 0)
    def _():
        s_ref[...] = jnp.dot(
            x_ref[...], w_ref[...], preferred_element_type=jnp.float32
        )

    bm = adj_refs[0].shape[0]
    for j in range(_NSTREAMS):
        acc = jnp.dot(
            adj_refs[j][...], s_ref[...], preferred_element_type=jnp.float32
        )
        out_ref[j * bm : (j + 1) * bm, :] = jnp.maximum(acc + b_ref[...], 0.0)


def kernel(x, adj, W, b):
    n, nfeat = x.shape
    nhid = W.shape[1]

    bm = _BM
    k = _NSTREAMS
    adj_specs = [
        pl.BlockSpec((bm, n), lambda i, j=j: (k * i + j, 0)) for j in range(k)
    ]
    out = pl.pallas_call(
        _gcn_kernel,
        grid=(n // (k * bm),),
        in_specs=adj_specs
        + [
            pl.BlockSpec((n, nfeat), lambda i: (0, 0)),
            pl.BlockSpec((nfeat, nhid), lambda i: (0, 0)),
            pl.BlockSpec((1, nhid), lambda i: (0, 0)),
        ],
        out_specs=pl.BlockSpec((k * bm, nhid), lambda i: (i, 0)),
        out_shape=jax.ShapeDtypeStruct((n, nhid), jnp.float32),
        scratch_shapes=[pltpu.VMEM((n, nhid), jnp.float32)],
    )(*([adj] * k), x, W, b.reshape(1, nhid))
    return out
